# Initial kernel scaffold; baseline (speedup 1.0000x reference)
import jax
import jax.numpy as jnp
from jax.experimental import pallas as pl


def kernel(x, edge_index, W1, b1, W2, b2, temp):
    raise NotImplementedError("write your pallas kernel here")


# speedup vs baseline: 3.7787x; 3.7787x over previous
import functools

import jax
import jax.numpy as jnp
from jax import lax
from jax.experimental import pallas as pl
from jax.experimental.pallas import tpu as pltpu
from jax.experimental.pallas import tpu_sc as plsc

N = 100000
D = 128
H = 64
C = 40
K = 10
CH = 20
CHW = 32
NP = 102400
NP2 = NP // 2
ACC_ROWS = NP2 + 64
ROWS_T = NP // 16
PH_ROWS_T = NP2 // 16
CHUNK = 160
NCHUNK = PH_ROWS_T // CHUNK
E = 1600000
EROWS = 12544
EP = EROWS * 128
EROWS_T = EROWS // 16
NWIN = EROWS_T // 2
NBLK = 50
BLK = NP // NBLK
FBLK = 2048
NFBLK = (N + FBLK - 1) // FBLK

_mesh = plsc.VectorSubcoreMesh(
    core_axis_name="c", subcore_axis_name="s", num_cores=2, num_subcores=16)


@functools.partial(
    pl.kernel,
    out_type=jax.ShapeDtypeStruct((2 * NP,), jnp.float32),
    mesh=_mesh,
    scratch_types=[
        pltpu.VMEM_SHARED((NP,), jnp.float32),
        pltpu.VMEM((8, 128), jnp.int32),
        pltpu.VMEM((6400,), jnp.float32),
        pltpu.VMEM((128,), jnp.float32),
    ],
)
def _deg_kernel(dst_hbm, degp_hbm, deg_sh, idx_v, zbuf, ones_v):
    c = lax.axis_index("c")
    s = lax.axis_index("s")
    rbase = s * ROWS_T

    def _zb(i, _):
        zbuf[pl.ds(i * 16, 16)] = jnp.zeros((16,), jnp.float32)
        return _
    lax.fori_loop(0, ROWS_T // 16, _zb, None)

    def _ob(i, _):
        ones_v[pl.ds(i * 16, 16)] = jnp.ones((16,), jnp.float32)
        return _
    lax.fori_loop(0, 8, _ob, None)

    pltpu.sync_copy(zbuf, deg_sh.at[pl.ds(rbase, ROWS_T)])
    plsc.subcore_barrier()

    wid = s * 2 + c
    row0 = wid * (EROWS // 32)

    def _win(w, _):
        base = row0 + w * 8
        pltpu.sync_copy(dst_hbm.at[pl.ds(base, 8)], idx_v)
        for j in range(8):
            pltpu.sync_copy(ones_v, deg_sh.at[idx_v.at[j]], add=True)
        return _
    lax.fori_loop(0, (EROWS // 32) // 8, _win, None)

    plsc.subcore_barrier()
    pltpu.sync_copy(deg_sh.at[pl.ds(rbase, ROWS_T)],
                    degp_hbm.at[pl.ds(c * NP + rbase, ROWS_T)])


@functools.partial(
    pl.kernel,
    out_type=(
        jax.ShapeDtypeStruct((2 * NP, CHW), jnp.float32),
        jax.ShapeDtypeStruct((2 * NP, CHW), jnp.float32),
        jax.ShapeDtypeStruct((2 * NP, CHW), jnp.float32),
    ),
    mesh=_mesh,
    scratch_types=[
        pltpu.VMEM_SHARED((ACC_ROWS, CHW), jnp.float32),
        pltpu.VMEM((128,), jnp.int32),
        pltpu.VMEM((128,), jnp.int32),
        pltpu.VMEM((128,), jnp.int32),
        pltpu.VMEM((128,), jnp.int32),
        pltpu.VMEM((128, CHW), jnp.float32),
        pltpu.VMEM((128, CHW), jnp.float32),
        pltpu.VMEM((CHUNK, CHW), jnp.float32),
        pltpu.VMEM((CHUNK, CHW), jnp.float32),
        pltpu.VMEM((PH_ROWS_T // 16, 16), jnp.float32),
        pltpu.VMEM((16,), jnp.float32),
        pltpu.SemaphoreType.DMA,
    ],
    compiler_params=pltpu.CompilerParams(use_tc_tiling_on_sc=False),
)
def _prop_kernel(u0_hbm, srcs_hbm, dst0_hbm, dst1_hbm, invdeg_hbm,
                 temp_hbm, hid_hbm, uta_hbm, utb_hbm, acc_sh,
                 ig0, ig1, ic0, ic1,
                 rv0, rv1, u_ch, hid_ch, invdeg_v, temp_v,
                 semg):
    igs = (ig0, ig1)
    ics = (ic0, ic1)
    rvs = (rv0, rv1)
    c = lax.axis_index("c")
    s = lax.axis_index("s")
    erow0 = s * EROWS_T

    PHV = PH_ROWS_T // 16
    pltpu.sync_copy(temp_hbm, temp_v)
    tv = temp_v[...]

    t0 = tv[0]

    def _pro(ci, _):
        r0g = s * ROWS_T + ci * CHUNK
        pltpu.sync_copy(u0_hbm.at[pl.ds(c * NP + r0g, CHUNK)], u_ch)

        def _rows(i, tk):
            r = i
            hid_ch[r, pl.ds(0, 16)] = tk * u_ch[r, pl.ds(0, 16)]
            hid_ch[r, pl.ds(8, 16)] = tk * u_ch[r, pl.ds(8, 16)]
            return tk
        lax.fori_loop(0, CHUNK, _rows, t0)
        pltpu.sync_copy(hid_ch, hid_hbm.at[pl.ds(c * NP + r0g, CHUNK)])
        return _
    lax.fori_loop(0, 2 * NCHUNK, _pro, None)
    plsc.subcore_barrier()

    for k in range(1, K + 1):
        rtab = u0_hbm if k == 1 else (uta_hbm if k % 2 == 0 else utb_hbm)
        wtab = uta_hbm if k % 2 == 1 else utb_hbm
        tk = tv[k]
        for p in range(2):
            dtab = dst0_hbm if p == 0 else dst1_hbm
            pltpu.sync_copy(
                invdeg_hbm.at[pl.ds((p * NP2 + s * PH_ROWS_T) // 16, PHV)],
                invdeg_v)
            def _init(ci, _):
                r0l = s * PH_ROWS_T + ci * CHUNK
                r0g = p * NP2 + r0l
                pltpu.sync_copy(rtab.at[pl.ds(c * NP + r0g, CHUNK)], u_ch)
                pltpu.sync_copy(u_ch, acc_sh.at[pl.ds(r0l, CHUNK)])
                return _
            lax.fori_loop(0, NCHUNK, _init, None)
            plsc.subcore_barrier()

            def _win(w, _):
                base = erow0 + w * 2
                for j in range(2):
                    pltpu.sync_copy(srcs_hbm.at[c, base + j], igs[j])
                    pltpu.sync_copy(dtab.at[base + j], ics[j])
                gs = [pltpu.async_copy(rtab.at[igs[j]], rvs[j], semg)
                      for j in range(2)]
                for g in gs:
                    g.wait()
                for j in range(2):
                    pltpu.sync_copy(rvs[j], acc_sh.at[ics[j]], add=True)
                return _
            lax.fori_loop(0, NWIN, _win, None)
            plsc.subcore_barrier()

            def _scale(ci, _):
                r0l = s * PH_ROWS_T + ci * CHUNK
                r0g = p * NP2 + r0l
                iv0 = ci * (CHUNK // 16)
                pltpu.sync_copy(acc_sh.at[pl.ds(r0l, CHUNK)], u_ch)
                pltpu.sync_copy(hid_hbm.at[pl.ds(c * NP + r0g, CHUNK)],
                                hid_ch)

                def _rows(i, t):
                    iv = invdeg_v[iv0 + i, :]
                    for q in range(16):
                        scl = iv[q]
                        r = i * 16 + q
                        u1 = u_ch[r, pl.ds(0, 16)] * scl
                        u2 = u_ch[r, pl.ds(8, 16)] * scl
                        ho1 = hid_ch[r, pl.ds(0, 16)]
                        ho2 = hid_ch[r, pl.ds(8, 16)]
                        u_ch[r, pl.ds(0, 16)] = u1
                        u_ch[r, pl.ds(8, 16)] = u2
                        hid_ch[r, pl.ds(0, 16)] = ho1 + t * u1
                        hid_ch[r, pl.ds(8, 16)] = ho2 + t * u2
                    return t
                lax.fori_loop(0, CHUNK // 16, _rows, tk)
                pltpu.sync_copy(u_ch, wtab.at[pl.ds(c * NP + r0g, CHUNK)])
                pltpu.sync_copy(hid_ch,
                                hid_hbm.at[pl.ds(c * NP + r0g, CHUNK)])
                return _
            lax.fori_loop(0, NCHUNK, _scale, None)
            plsc.subcore_barrier()


def _mlp_body(x_ref, degp_ref, W1_ref, b1_ref, W2_ref, b2_ref,
              u0_ref, invdeg_ref, sqdeg_ref):
    xb = x_ref[...]
    h = jnp.maximum(jnp.dot(xb, W1_ref[...].T,
                            preferred_element_type=jnp.float32)
                    + b1_ref[...][None, :], 0.0)
    h = jnp.dot(h, W2_ref[...].T,
                preferred_element_type=jnp.float32) + b2_ref[...][None, :]
    deg = 1.0 + degp_ref[0] + degp_ref[1]
    dis = lax.rsqrt(deg)
    u0 = h * dis[:, None]
    zp = jnp.zeros((u0.shape[0], CHW - CH), jnp.float32)
    u0_ref[0] = jnp.concatenate([u0[:, :CH], zp], axis=1)
    u0_ref[1] = jnp.concatenate([u0[:, CH:], zp], axis=1)
    invdeg_ref[0, 0] = 1.0 / deg
    sqdeg_ref[0, 0] = deg * dis


def _mlp(x, degp, W1, b1, W2, b2):
    return pl.pallas_call(
        _mlp_body,
        grid=(NBLK,),
        in_specs=[
            pl.BlockSpec((BLK, D), lambda i: (i, 0)),
            pl.BlockSpec((2, BLK), lambda i: (0, i)),
            pl.BlockSpec((H, D), lambda i: (0, 0)),
            pl.BlockSpec((H,), lambda i: (0,)),
            pl.BlockSpec((C, H), lambda i: (0, 0)),
            pl.BlockSpec((C,), lambda i: (0,)),
        ],
        out_specs=[
            pl.BlockSpec((2, BLK, CHW), lambda i: (0, i, 0)),
            pl.BlockSpec((1, 1, BLK), lambda i: (i, 0, 0)),
            pl.BlockSpec((1, 1, BLK), lambda i: (i, 0, 0)),
        ],
        out_shape=[
            jax.ShapeDtypeStruct((2, NP, CHW), jnp.float32),
            jax.ShapeDtypeStruct((NBLK, 1, BLK), jnp.float32),
            jax.ShapeDtypeStruct((NBLK, 1, BLK), jnp.float32),
        ],
    )(x, degp, W1, b1, W2, b2)


def _fin_body(hid_ref, sq_ref, out_ref):
    hu = hid_ref[...]
    sq = sq_ref[0, 0]
    h = jnp.concatenate([hu[0][:, :CH], hu[1][:, :CH]], axis=1) * sq[:, None]
    m = jnp.max(h, axis=1, keepdims=True)
    e = h - m
    out_ref[...] = e - jnp.log(jnp.sum(jnp.exp(e), axis=1, keepdims=True))


def _final(hid, sq2d):
    return pl.pallas_call(
        _fin_body,
        grid=(NFBLK,),
        in_specs=[
            pl.BlockSpec((2, FBLK, CHW), lambda i: (0, i, 0)),
            pl.BlockSpec((1, 1, FBLK), lambda i: (i, 0, 0)),
        ],
        out_specs=pl.BlockSpec((FBLK, C), lambda i: (i, 0)),
        out_shape=jax.ShapeDtypeStruct((N, C), jnp.float32),
    )(hid, sq2d)


def kernel(x, edge_index, W1, b1, W2, b2, temp):
    src = edge_index[0]
    dst = edge_index[1]
    npad = EP - E
    pad_src = jnp.zeros((npad,), jnp.int32)
    pad_dst = N + (jnp.arange(npad, dtype=jnp.int32) % 2048)
    src_p = jnp.concatenate([src, pad_src]).reshape(EROWS, 128)
    dst_p = jnp.concatenate([dst, pad_dst]).reshape(EROWS, 128)
    src1_p = src_p + NP
    trash = NP2 + (dst_p & 63)
    dst0_p = jnp.where(dst_p < NP2, dst_p, trash)
    dst1_p = jnp.where(dst_p >= NP2, dst_p - NP2, trash)
    temp_p = jnp.concatenate([temp, jnp.zeros((16 - K - 1,), jnp.float32)])

    degp = _deg_kernel(dst_p).reshape(2, NP)
    x_p = jnp.pad(x, ((0, NP - N), (0, 0)))
    u0n, invdeg2d, sqdeg2d = _mlp(x_p, degp, W1, b1, W2, b2)
    u0 = u0n.reshape(2 * NP, CHW)
    invdeg = invdeg2d.reshape(NP // 16, 16)

    srcs_p = jnp.stack([src_p, src1_p])
    hid, _, _ = _prop_kernel(u0, srcs_p, dst0_p, dst1_p, invdeg, temp_p)
    return _final(hid.reshape(2, NP, CHW), sqdeg2d)
